# initial kernel scaffold (unmeasured)
import jax
import jax.numpy as jnp
from jax import lax
from jax.experimental import pallas as pl
from jax.experimental.pallas import tpu as pltpu

M = 4096
D = 4096


def _exchange_y(p_mine):

    def body(p_ref, other_ref, send_sem, recv_sem):
        my_x = lax.axis_index("x")
        my_y = lax.axis_index("y")
        my_z = lax.axis_index("z")
        peer = (my_x, 1 - my_y, my_z)

        barrier = pltpu.get_barrier_semaphore()
        pl.semaphore_signal(
            barrier, inc=1, device_id=peer, device_id_type=pl.DeviceIdType.MESH
        )
        pl.semaphore_wait(barrier, 1)

        rdma = pltpu.make_async_remote_copy(
            src_ref=p_ref,
            dst_ref=other_ref,
            send_sem=send_sem,
            recv_sem=recv_sem,
            device_id=peer,
            device_id_type=pl.DeviceIdType.MESH,
        )
        rdma.start()
        rdma.wait()

    return pl.pallas_call(
        body,
        out_shape=jax.ShapeDtypeStruct((M, D), jnp.float32),
        in_specs=[pl.BlockSpec(memory_space=pltpu.ANY)],
        out_specs=pl.BlockSpec(memory_space=pltpu.ANY),
        scratch_shapes=[
            pltpu.SemaphoreType.DMA,
            pltpu.SemaphoreType.DMA,
        ],
        compiler_params=pltpu.CompilerParams(collective_id=0),
    )(p_mine)


def _ln_resid(p_mine, p_other, resid, gamma2d):
    BLK = 256

    def body(a_ref, b_ref, r_ref, g_ref, out_ref):
        y = a_ref[...] + b_ref[...] + r_ref[...]
        ms = jnp.mean(y * y, axis=-1, keepdims=True)
        out_ref[...] = y * lax.rsqrt(ms + 1e-6) * g_ref[...]

    return pl.pallas_call(
        body,
        grid=(M // BLK,),
        in_specs=[
            pl.BlockSpec((BLK, D), lambda i: (i, 0)),
            pl.BlockSpec((BLK, D), lambda i: (i, 0)),
            pl.BlockSpec((BLK, D), lambda i: (i, 0)),
            pl.BlockSpec((1, D), lambda i: (0, 0)),
        ],
        out_specs=pl.BlockSpec((BLK, D), lambda i: (i, 0)),
        out_shape=jax.ShapeDtypeStruct((M, D), jnp.float32),
    )(p_mine, p_other, resid, gamma2d)


def kernel(partial, resid, gamma):
    p_mine = partial.reshape(M, D)
    p_other = _exchange_y(p_mine)
    return _ln_resid(p_mine, p_other, resid, gamma.reshape(1, D))


# baseline (device time: 806917 ns/iter reference)
import jax
import jax.numpy as jnp
from jax import lax
from jax.experimental import pallas as pl
from jax.experimental.pallas import tpu as pltpu

M = 4096
D = 4096


def _exchange_y(p_mine):

    def body(p_ref, other_ref, send_sem, recv_sem):
        my_x = lax.axis_index("x")
        my_y = lax.axis_index("y")
        my_z = lax.axis_index("z")
        peer = (my_x, 1 - my_y, my_z)

        barrier = pltpu.get_barrier_semaphore()
        pl.semaphore_signal(
            barrier, inc=1, device_id=peer, device_id_type=pl.DeviceIdType.MESH
        )
        pl.semaphore_wait(barrier, 1)

        rdma = pltpu.make_async_remote_copy(
            src_ref=p_ref,
            dst_ref=other_ref,
            send_sem=send_sem,
            recv_sem=recv_sem,
            device_id=peer,
            device_id_type=pl.DeviceIdType.MESH,
        )
        rdma.start()
        rdma.wait()

    return pl.pallas_call(
        body,
        out_shape=jax.ShapeDtypeStruct((M, D), jnp.float32),
        in_specs=[pl.BlockSpec(memory_space=pl.ANY)],
        out_specs=pl.BlockSpec(memory_space=pl.ANY),
        scratch_shapes=[
            pltpu.SemaphoreType.DMA,
            pltpu.SemaphoreType.DMA,
        ],
        compiler_params=pltpu.CompilerParams(collective_id=0),
    )(p_mine)


def _ln_resid(p_mine, p_other, resid, gamma2d):
    BLK = 128

    def body(a_ref, b_ref, r_ref, g_ref, out_ref):
        y = a_ref[...] + b_ref[...] + r_ref[...]
        ms = jnp.mean(y * y, axis=-1, keepdims=True)
        out_ref[...] = y * lax.rsqrt(ms + 1e-6) * g_ref[...]

    return pl.pallas_call(
        body,
        grid=(M // BLK,),
        in_specs=[
            pl.BlockSpec((BLK, D), lambda i: (i, 0)),
            pl.BlockSpec((BLK, D), lambda i: (i, 0)),
            pl.BlockSpec((BLK, D), lambda i: (i, 0)),
            pl.BlockSpec((1, D), lambda i: (0, 0)),
        ],
        out_specs=pl.BlockSpec((BLK, D), lambda i: (i, 0)),
        out_shape=jax.ShapeDtypeStruct((M, D), jnp.float32),
    )(p_mine, p_other, resid, gamma2d)


def kernel(partial, resid, gamma):
    p_mine = partial.reshape(M, D)
    p_other = _exchange_y(p_mine)
    return _ln_resid(p_mine, p_other, resid, gamma.reshape(1, D))


# device time: 473519 ns/iter; 1.7041x vs baseline; 1.7041x over previous
import jax
import jax.numpy as jnp
from jax import lax
from jax.experimental import pallas as pl
from jax.experimental.pallas import tpu as pltpu

M = 4096
D = 4096


HALF = M // 2
CH = 128
C = HALF // CH


def _exchange_y(p_mine):

    def body(p_ref, other_ref, y_send, y_recv, x_send, x_recv):
        my_x = lax.axis_index("x")
        my_y = lax.axis_index("y")
        my_z = lax.axis_index("z")
        y_peer = (my_x, 1 - my_y, my_z)
        x_peer = (1 - my_x, my_y, my_z)

        barrier = pltpu.get_barrier_semaphore()
        for peer in (y_peer, x_peer):
            pl.semaphore_signal(
                barrier, inc=1, device_id=peer,
                device_id_type=pl.DeviceIdType.MESH,
            )
        pl.semaphore_wait(barrier, 2)

        my_half = my_x * HALF
        other_half = (1 - my_x) * HALF

        y_rdmas = []
        for c in range(C):
            rows = pl.ds(my_half + c * CH, CH)
            r = pltpu.make_async_remote_copy(
                src_ref=p_ref.at[rows, :],
                dst_ref=other_ref.at[rows, :],
                send_sem=y_send.at[c],
                recv_sem=y_recv.at[c],
                device_id=y_peer,
                device_id_type=pl.DeviceIdType.MESH,
            )
            r.start()
            y_rdmas.append(r)

        x_rdmas = []
        for c in range(C):
            rows = pl.ds(my_half + c * CH, CH)
            y_rdmas[c].wait_recv()
            r = pltpu.make_async_remote_copy(
                src_ref=other_ref.at[rows, :],
                dst_ref=other_ref.at[rows, :],
                send_sem=x_send.at[c],
                recv_sem=x_recv.at[c],
                device_id=x_peer,
                device_id_type=pl.DeviceIdType.MESH,
            )
            r.start()
            x_rdmas.append(r)

        for c in range(C):
            rows = pl.ds(other_half + c * CH, CH)
            rr = pltpu.make_async_remote_copy(
                src_ref=other_ref.at[rows, :],
                dst_ref=other_ref.at[rows, :],
                send_sem=x_send.at[c],
                recv_sem=x_recv.at[c],
                device_id=x_peer,
                device_id_type=pl.DeviceIdType.MESH,
            )
            rr.wait_recv()

        for c in range(C):
            y_rdmas[c].wait_send()
            x_rdmas[c].wait_send()

    return pl.pallas_call(
        body,
        out_shape=jax.ShapeDtypeStruct((M, D), jnp.float32),
        in_specs=[pl.BlockSpec(memory_space=pl.ANY)],
        out_specs=pl.BlockSpec(memory_space=pl.ANY),
        scratch_shapes=[
            pltpu.SemaphoreType.DMA((C,)),
            pltpu.SemaphoreType.DMA((C,)),
            pltpu.SemaphoreType.DMA((C,)),
            pltpu.SemaphoreType.DMA((C,)),
        ],
        compiler_params=pltpu.CompilerParams(collective_id=0),
    )(p_mine)


def _ln_resid(p_mine, p_other, resid, gamma2d):
    BLK = 128

    def body(a_ref, b_ref, r_ref, g_ref, out_ref):
        y = a_ref[...] + b_ref[...] + r_ref[...]
        ms = jnp.mean(y * y, axis=-1, keepdims=True)
        out_ref[...] = y * lax.rsqrt(ms + 1e-6) * g_ref[...]

    return pl.pallas_call(
        body,
        grid=(M // BLK,),
        in_specs=[
            pl.BlockSpec((BLK, D), lambda i: (i, 0)),
            pl.BlockSpec((BLK, D), lambda i: (i, 0)),
            pl.BlockSpec((BLK, D), lambda i: (i, 0)),
            pl.BlockSpec((1, D), lambda i: (0, 0)),
        ],
        out_specs=pl.BlockSpec((BLK, D), lambda i: (i, 0)),
        out_shape=jax.ShapeDtypeStruct((M, D), jnp.float32),
    )(p_mine, p_other, resid, gamma2d)


def kernel(partial, resid, gamma):
    p_mine = partial.reshape(M, D)
    p_other = _exchange_y(p_mine)
    return _ln_resid(p_mine, p_other, resid, gamma.reshape(1, D))


# device time: 440143 ns/iter; 1.8333x vs baseline; 1.0758x over previous
import jax
import jax.numpy as jnp
from jax import lax
from jax.experimental import pallas as pl
from jax.experimental.pallas import tpu as pltpu

M = 4096
D = 4096
HALF = M // 2
CH = 128
C = HALF // CH


def _fused(p_mine, resid, gamma2d):
    def body(
        p_ref, r_ref, g_ref,
        out_ref, other_ref,
        a_vm, b_vm, rs_vm, o_vm,
        y_send, y_recv, x_send, x_recv,
        a_sem, b_sem, rs_sem, out_sem,
    ):
        my_x = lax.axis_index("x")
        my_y = lax.axis_index("y")
        my_z = lax.axis_index("z")
        y_peer = (my_x, 1 - my_y, my_z)
        x_peer = (1 - my_x, my_y, my_z)

        barrier = pltpu.get_barrier_semaphore()
        for peer in (y_peer, x_peer):
            pl.semaphore_signal(
                barrier, inc=1, device_id=peer,
                device_id_type=pl.DeviceIdType.MESH,
            )
        pl.semaphore_wait(barrier, 2)

        my_half = my_x * HALF
        other_half = (1 - my_x) * HALF

        y_rdmas = []
        for c in range(C):
            rows = pl.ds(my_half + c * CH, CH)
            r = pltpu.make_async_remote_copy(
                src_ref=p_ref.at[rows, :],
                dst_ref=other_ref.at[rows, :],
                send_sem=y_send.at[c],
                recv_sem=y_recv.at[c],
                device_id=y_peer,
                device_id_type=pl.DeviceIdType.MESH,
            )
            r.start()
            y_rdmas.append(r)

        x_rdmas = []
        out_cps = []

        def stage_and_compute(k, rows):
            s = k % 2
            if k >= 2:
                out_cps[k - 2].wait()
            a_cp = pltpu.make_async_copy(p_ref.at[rows, :], a_vm.at[s], a_sem.at[s])
            b_cp = pltpu.make_async_copy(other_ref.at[rows, :], b_vm.at[s], b_sem.at[s])
            r_cp = pltpu.make_async_copy(r_ref.at[rows, :], rs_vm.at[s], rs_sem.at[s])
            a_cp.start(); b_cp.start(); r_cp.start()
            a_cp.wait(); b_cp.wait(); r_cp.wait()
            y = a_vm[s, :, :] + b_vm[s, :, :] + rs_vm[s, :, :]
            ms = jnp.mean(y * y, axis=-1, keepdims=True)
            o_vm[s, :, :] = y * lax.rsqrt(ms + 1e-6) * g_ref[...]
            o_cp = pltpu.make_async_copy(o_vm.at[s], out_ref.at[rows, :], out_sem.at[s])
            o_cp.start()
            out_cps.append(o_cp)

        k = 0
        for c in range(C):
            my_rows = pl.ds(my_half + c * CH, CH)
            y_rdmas[c].wait_recv()
            fwd = pltpu.make_async_remote_copy(
                src_ref=other_ref.at[my_rows, :],
                dst_ref=other_ref.at[my_rows, :],
                send_sem=x_send.at[c],
                recv_sem=x_recv.at[c],
                device_id=x_peer,
                device_id_type=pl.DeviceIdType.MESH,
            )
            fwd.start()
            x_rdmas.append(fwd)

            stage_and_compute(k, my_rows)
            k += 1

            if c >= 1:
                o_rows = pl.ds(other_half + (c - 1) * CH, CH)
                rr = pltpu.make_async_remote_copy(
                    src_ref=other_ref.at[o_rows, :],
                    dst_ref=other_ref.at[o_rows, :],
                    send_sem=x_send.at[c - 1],
                    recv_sem=x_recv.at[c - 1],
                    device_id=x_peer,
                    device_id_type=pl.DeviceIdType.MESH,
                )
                rr.wait_recv()
                stage_and_compute(k, o_rows)
                k += 1

        o_rows = pl.ds(other_half + (C - 1) * CH, CH)
        rr = pltpu.make_async_remote_copy(
            src_ref=other_ref.at[o_rows, :],
            dst_ref=other_ref.at[o_rows, :],
            send_sem=x_send.at[C - 1],
            recv_sem=x_recv.at[C - 1],
            device_id=x_peer,
            device_id_type=pl.DeviceIdType.MESH,
        )
        rr.wait_recv()
        stage_and_compute(k, o_rows)
        k += 1

        for c in range(C):
            y_rdmas[c].wait_send()
            x_rdmas[c].wait_send()
        out_cps[k - 2].wait()
        out_cps[k - 1].wait()

    out, _ = pl.pallas_call(
        body,
        out_shape=[
            jax.ShapeDtypeStruct((M, D), jnp.float32),
            jax.ShapeDtypeStruct((M, D), jnp.float32),
        ],
        in_specs=[
            pl.BlockSpec(memory_space=pl.ANY),
            pl.BlockSpec(memory_space=pl.ANY),
            pl.BlockSpec(memory_space=pltpu.MemorySpace.VMEM),
        ],
        out_specs=[
            pl.BlockSpec(memory_space=pl.ANY),
            pl.BlockSpec(memory_space=pl.ANY),
        ],
        scratch_shapes=[
            pltpu.VMEM((2, CH, D), jnp.float32),
            pltpu.VMEM((2, CH, D), jnp.float32),
            pltpu.VMEM((2, CH, D), jnp.float32),
            pltpu.VMEM((2, CH, D), jnp.float32),
            pltpu.SemaphoreType.DMA((C,)),
            pltpu.SemaphoreType.DMA((C,)),
            pltpu.SemaphoreType.DMA((C,)),
            pltpu.SemaphoreType.DMA((C,)),
            pltpu.SemaphoreType.DMA((2,)),
            pltpu.SemaphoreType.DMA((2,)),
            pltpu.SemaphoreType.DMA((2,)),
            pltpu.SemaphoreType.DMA((2,)),
        ],
        compiler_params=pltpu.CompilerParams(collective_id=0),
    )(p_mine, resid, gamma2d)
    return out


def kernel(partial, resid, gamma):
    p_mine = partial.reshape(M, D)
    return _fused(p_mine, resid, gamma.reshape(1, D))
